# baseline (speedup 1.0000x reference)
import jax
import jax.numpy as jnp
from jax.experimental import pallas as pl

_N = 1024
_NHID = 32
_NHEADS = 4
_LOG2E = 1.4426950408889634


def _layernorm(x, eps=1e-5):
    m = jnp.mean(x, axis=-1, keepdims=True)
    v = jnp.var(x, axis=-1, keepdims=True)
    return (x - m) / jnp.sqrt(v + eps)


def _elu(x):
    return jnp.where(x > 0, x, jnp.exp(x) - 1.0)


def _gat_layer(x, mask, W_ref, a_ref, concat):
    Wcat = jnp.concatenate([W_ref[i] for i in range(_NHEADS)], axis=1)
    h_all = jnp.dot(x, Wcat, preferred_element_type=jnp.float32)
    ones = jnp.ones((_N, 1), dtype=jnp.float32)
    outs = []
    for i in range(_NHEADS):
        h = h_all[:, i * _NHID:(i + 1) * _NHID]
        a = a_ref[i, 0]
        us = jnp.sum(h * (-_LOG2E * a[:_NHID])[None, :], axis=1, keepdims=True)
        ud = jnp.sum(h * (-_LOG2E * a[_NHID:])[None, :], axis=1, keepdims=True)
        u = us + jnp.transpose(ud)
        E = jnp.exp2(jnp.minimum(u, 0.2 * u)) * mask
        h_aug = jnp.concatenate([h, ones], axis=1)
        nd = jnp.dot(E, h_aug, preferred_element_type=jnp.float32)
        hp = nd[:, :_NHID] / nd[:, _NHID:_NHID + 1]
        outs.append(_elu(hp) if concat else hp)
    return jnp.concatenate(outs, axis=1)


def _gat_body(x_in_ref, adj_ref, emb_ref, W1_ref, a1_ref, W2_ref, a2_ref,
              Wf_ref, af_ref, Wout_ref, bout_ref, out_ref):
    mask = adj_ref[...].astype(jnp.float32)
    x = jnp.dot(x_in_ref[...], emb_ref[...], preferred_element_type=jnp.float32)
    x = _layernorm(x)
    x = _layernorm(_gat_layer(x, mask, W1_ref, a1_ref, True))
    x = _layernorm(_gat_layer(x, mask, W2_ref, a2_ref, True))
    x = _layernorm(_gat_layer(x, mask, Wf_ref, af_ref, False))
    x = _elu(x)
    logits = jnp.dot(x, Wout_ref[...], preferred_element_type=jnp.float32)
    logits = logits + bout_ref[...][None, :]
    m = jnp.max(logits, axis=1, keepdims=True)
    s = logits - m
    lse = jnp.log(jnp.sum(jnp.exp(s), axis=1, keepdims=True))
    out_ref[...] = s - lse


def kernel(x_in, adj, emb, W1, a1, W2, a2, Wf, af, Wout, bout):
    return pl.pallas_call(
        _gat_body,
        out_shape=jax.ShapeDtypeStruct((_N, 40), jnp.float32),
    )(x_in, adj, emb, W1, a1, W2, a2, Wf, af, Wout, bout)


# speedup vs baseline: 1.1101x; 1.1101x over previous
import jax
import jax.numpy as jnp
from jax.experimental import pallas as pl

_N = 1024
_NHID = 32
_NHEADS = 4
_LOG2E = 1.4426950408889634


def _layernorm(x, eps=1e-5):
    m = jnp.mean(x, axis=-1, keepdims=True)
    v = jnp.var(x, axis=-1, keepdims=True)
    return (x - m) / jnp.sqrt(v + eps)


def _elu(x):
    return jnp.where(x > 0, x, jnp.exp(x) - 1.0)


def _head_logits(h_all, a_ref):
    asrc = jnp.concatenate(
        [(-_LOG2E) * a_ref[i, 0, :_NHID] for i in range(_NHEADS)])
    adst = jnp.concatenate(
        [(-_LOG2E) * a_ref[i, 0, _NHID:] for i in range(_NHEADS)])
    dim = _NHEADS * _NHID
    sel = (jax.lax.broadcasted_iota(jnp.int32, (dim, _NHEADS), 0) // _NHID
           == jax.lax.broadcasted_iota(jnp.int32, (dim, _NHEADS), 1)
           ).astype(jnp.float32)
    us = jnp.dot(h_all * asrc[None, :], sel, preferred_element_type=jnp.float32)
    ud = jnp.dot(h_all * adst[None, :], sel, preferred_element_type=jnp.float32)
    return us, jnp.transpose(ud)


def _gat_layer(x, mask, W_ref, a_ref, concat):
    Wcat = jnp.concatenate([W_ref[i] for i in range(_NHEADS)], axis=1)
    h_all = jnp.dot(x, Wcat, preferred_element_type=jnp.float32)
    us_all, udT_all = _head_logits(h_all, a_ref)
    ones = jnp.ones((_N, 1), dtype=jnp.float32)
    outs = []
    for i in range(_NHEADS):
        h = h_all[:, i * _NHID:(i + 1) * _NHID]
        u = us_all[:, i:i + 1] + udT_all[i:i + 1, :]
        E = jnp.exp2(jnp.minimum(u, 0.2 * u)) * mask
        h_aug = jnp.concatenate([h, ones], axis=1)
        nd = jnp.dot(E, h_aug, preferred_element_type=jnp.float32)
        hp = nd[:, :_NHID] / nd[:, _NHID:_NHID + 1]
        outs.append(_elu(hp) if concat else hp)
    return jnp.concatenate(outs, axis=1)


def _gat_body(x_in_ref, adj_ref, emb_ref, W1_ref, a1_ref, W2_ref, a2_ref,
              Wf_ref, af_ref, Wout_ref, bout_ref, out_ref):
    mask = adj_ref[...].astype(jnp.float32)
    x = jnp.dot(x_in_ref[...], emb_ref[...], preferred_element_type=jnp.float32)
    x = _layernorm(x)
    x = _layernorm(_gat_layer(x, mask, W1_ref, a1_ref, True))
    x = _layernorm(_gat_layer(x, mask, W2_ref, a2_ref, True))
    x = _layernorm(_gat_layer(x, mask, Wf_ref, af_ref, False))
    x = _elu(x)
    logits = jnp.dot(x, Wout_ref[...], preferred_element_type=jnp.float32)
    logits = logits + bout_ref[...][None, :]
    m = jnp.max(logits, axis=1, keepdims=True)
    s = logits - m
    lse = jnp.log(jnp.sum(jnp.exp(s), axis=1, keepdims=True))
    out_ref[...] = s - lse


def kernel(x_in, adj, emb, W1, a1, W2, a2, Wf, af, Wout, bout):
    return pl.pallas_call(
        _gat_body,
        out_shape=jax.ShapeDtypeStruct((_N, 40), jnp.float32),
    )(x_in, adj, emb, W1, a1, W2, a2, Wf, af, Wout, bout)


# speedup vs baseline: 1.2089x; 1.0890x over previous
import jax
import jax.numpy as jnp
from jax.experimental import pallas as pl

_N = 1024
_NHID = 32
_NHEADS = 4
_LOG2E = 1.4426950408889634


def _layernorm(x, eps=1e-5):
    m = jnp.mean(x, axis=-1, keepdims=True)
    v = jnp.var(x, axis=-1, keepdims=True)
    return (x - m) / jnp.sqrt(v + eps)


def _elu(x):
    return jnp.where(x > 0, x, jnp.exp(x) - 1.0)


def _head_logits(h_all, a_ref):
    asrc = jnp.concatenate(
        [(-_LOG2E) * a_ref[i, 0, :_NHID] for i in range(_NHEADS)])
    adst = jnp.concatenate(
        [(-_LOG2E) * a_ref[i, 0, _NHID:] for i in range(_NHEADS)])
    dim = _NHEADS * _NHID
    sel = (jax.lax.broadcasted_iota(jnp.int32, (dim, _NHEADS), 0) // _NHID
           == jax.lax.broadcasted_iota(jnp.int32, (dim, _NHEADS), 1)
           ).astype(jnp.float32)
    us = jnp.dot(h_all * asrc[None, :], sel, preferred_element_type=jnp.float32)
    ud = jnp.dot(h_all * adst[None, :], sel, preferred_element_type=jnp.float32)
    return us, jnp.transpose(ud)


def _gat_layer(x, mask, W_ref, a_ref, concat):
    Wcat = jnp.concatenate([W_ref[i] for i in range(_NHEADS)], axis=1)
    h_all = jnp.dot(x, Wcat, preferred_element_type=jnp.float32)
    us_all, udT_all = _head_logits(h_all, a_ref)
    gs_all = jnp.exp2(-0.8 * us_all)
    edT_all = jnp.exp2(udT_all)
    fdT_all = jnp.exp2(0.2 * udT_all)
    ones = jnp.ones((_N, 1), dtype=jnp.float32)
    outs = []
    for i in range(_NHEADS):
        h = h_all[:, i * _NHID:(i + 1) * _NHID]
        E = jnp.minimum(edT_all[i:i + 1, :],
                        gs_all[:, i:i + 1] * fdT_all[i:i + 1, :]) * mask
        h_aug = jnp.concatenate([h, ones], axis=1)
        nd = jnp.dot(E, h_aug, preferred_element_type=jnp.float32)
        hp = nd[:, :_NHID] / nd[:, _NHID:_NHID + 1]
        outs.append(_elu(hp) if concat else hp)
    return jnp.concatenate(outs, axis=1)


def _gat_body(x_in_ref, adj_ref, emb_ref, W1_ref, a1_ref, W2_ref, a2_ref,
              Wf_ref, af_ref, Wout_ref, bout_ref, out_ref):
    mask = adj_ref[...].astype(jnp.float32)
    x = jnp.dot(x_in_ref[...], emb_ref[...], preferred_element_type=jnp.float32)
    x = _layernorm(x)
    x = _layernorm(_gat_layer(x, mask, W1_ref, a1_ref, True))
    x = _layernorm(_gat_layer(x, mask, W2_ref, a2_ref, True))
    x = _layernorm(_gat_layer(x, mask, Wf_ref, af_ref, False))
    x = _elu(x)
    logits = jnp.dot(x, Wout_ref[...], preferred_element_type=jnp.float32)
    logits = logits + bout_ref[...][None, :]
    m = jnp.max(logits, axis=1, keepdims=True)
    s = logits - m
    lse = jnp.log(jnp.sum(jnp.exp(s), axis=1, keepdims=True))
    out_ref[...] = s - lse


def kernel(x_in, adj, emb, W1, a1, W2, a2, Wf, af, Wout, bout):
    return pl.pallas_call(
        _gat_body,
        out_shape=jax.ShapeDtypeStruct((_N, 40), jnp.float32),
    )(x_in, adj, emb, W1, a1, W2, a2, Wf, af, Wout, bout)
